# baseline (speedup 1.0000x reference)
import dataclasses
import functools

import jax
import jax.numpy as jnp
from jax import lax
from jax.experimental import pallas as pl
from jax.experimental.pallas import tpu as pltpu
from jax.experimental.pallas import tpu_sc as plsc

L = 16
NC = 128
BP = 2


def kernel(t, permutations):
    b, h, n, d = t.shape
    perms = permutations[:, :n]

    tT = jnp.transpose(t, (0, 1, 3, 2))
    pT = jnp.transpose(perms, (0, 2, 1))

    mesh = plsc.VectorSubcoreMesh(core_axis_name="c", subcore_axis_name="s")
    cp = pltpu.CompilerParams()
    if "needs_layout_passes" in pltpu.CompilerParams.__dataclass_fields__:
        cp = dataclasses.replace(cp, needs_layout_passes=False)
    if "use_tc_tiling_on_sc" in pltpu.CompilerParams.__dataclass_fields__:
        cp = dataclasses.replace(cp, use_tc_tiling_on_sc=True)

    @functools.partial(
        pl.kernel,
        out_type=jax.ShapeDtypeStruct(tT.shape, tT.dtype),
        mesh=mesh,
        compiler_params=cp,
    )
    def run(t_hbm, p_hbm, o_hbm):
        def body(t_v, p_v, o_v):
            cols = [lax.iota(jnp.int32, L) + q * L for q in range(NC // L)]

            @plsc.parallel_loop(0, d, unroll=4)
            def _(j):
                for q in range(NC // L):
                    rows = p_v[0, j, pl.ds(q * L, L)]
                    for bb in range(BP):
                        vals = plsc.load_gather(t_v.at[bb, 0], [rows, cols[q]])
                        o_v[bb, 0, j, pl.ds(q * L, L)] = vals

        pltpu.emit_pipeline(
            body,
            grid=(h, n // NC, b // BP),
            in_specs=[
                pl.BlockSpec((BP, 1, d, NC), lambda i, j, k: (k, i, 0, j)),
                pl.BlockSpec((1, d, NC), lambda i, j, k: (i, 0, j)),
            ],
            out_specs=[
                pl.BlockSpec((BP, 1, d, NC), lambda i, j, k: (k, i, 0, j)),
            ],
            core_axis_name=("c", "s"),
            dimension_semantics=(pltpu.PARALLEL, pltpu.PARALLEL, pltpu.PARALLEL),
        )(t_hbm, p_hbm, o_hbm)

    return jnp.transpose(run(tT, pT), (0, 1, 3, 2))
